# Initial kernel scaffold; baseline (speedup 1.0000x reference)
import jax
import jax.numpy as jnp
from jax.experimental import pallas as pl


def kernel(x, edge_index, time_step, W1, as1, ad1, b1g, W2, as2, ad2, b2g, Wqkv, bqkv, Wo, bo, ln_g, ln_b, Wf1, bf1, Wf2, bf2, Wc, bc):
    raise NotImplementedError("write your pallas kernel here")


# speedup vs baseline: 4.3088x; 4.3088x over previous
import functools
import numpy as np
import jax
import jax.numpy as jnp
from jax.experimental import pallas as pl

N = 10000
D_IN = 128
HID = 128
OUT = 128
HEADS = 4
L = 5
NCLS = 2

BN = 1000


def _make_pe_row(d, pos):
    pe = np.zeros((d,), dtype=np.float32)
    div = np.exp(np.arange(0, d, 2, dtype=np.float32) * (-np.log(10000.0) / d))
    pe[0::2] = np.sin(pos * div)
    pe[1::2] = np.cos(pos * div)
    return pe

_PE4 = _make_pe_row(OUT, float(L - 1))


def _ka_body(x_ref, w_ref, as_ref, ad_ref, h0_ref, als_ref, ald_ref):
    h0 = jnp.dot(x_ref[...], w_ref[...], preferred_element_type=jnp.float32)
    h0_ref[...] = h0
    als = []
    ald = []
    for h in range(HEADS):
        sl = h0[:, h * OUT:(h + 1) * OUT]
        als.append((sl * as_ref[h, :][None, :]).sum(-1, keepdims=True))
        ald.append((sl * ad_ref[h, :][None, :]).sum(-1, keepdims=True))
    als_ref[...] = jnp.concatenate(als, axis=-1)
    ald_ref[...] = jnp.concatenate(ald, axis=-1)


def _stage_a(x, W, a_s, a_d, d_in, d_out):
    return pl.pallas_call(
        _ka_body,
        grid=(N // BN,),
        in_specs=[
            pl.BlockSpec((BN, d_in), lambda i: (i, 0)),
            pl.BlockSpec((d_in, d_out), lambda i: (0, 0)),
            pl.BlockSpec((HEADS, OUT), lambda i: (0, 0)),
            pl.BlockSpec((HEADS, OUT), lambda i: (0, 0)),
        ],
        out_specs=[
            pl.BlockSpec((BN, d_out), lambda i: (i, 0)),
            pl.BlockSpec((BN, HEADS), lambda i: (i, 0)),
            pl.BlockSpec((BN, HEADS), lambda i: (i, 0)),
        ],
        out_shape=[
            jax.ShapeDtypeStruct((N, d_out), jnp.float32),
            jax.ShapeDtypeStruct((N, HEADS), jnp.float32),
            jax.ShapeDtypeStruct((N, HEADS), jnp.float32),
        ],
    )(x, W, a_s, a_d)


def _kb_body(agg_ref, b_ref, w_ref, as_ref, ad_ref, h_ref, h1_ref, als_ref, ald_ref):
    a = agg_ref[...] + b_ref[...]
    h = jnp.where(a > 0, a, jnp.exp(jnp.minimum(a, 0.0)) - 1.0)
    h_ref[...] = h
    h1 = jnp.dot(h, w_ref[...], preferred_element_type=jnp.float32)
    h1_ref[...] = h1
    als = []
    ald = []
    for hh in range(HEADS):
        sl = h1[:, hh * OUT:(hh + 1) * OUT]
        als.append((sl * as_ref[hh, :][None, :]).sum(-1, keepdims=True))
        ald.append((sl * ad_ref[hh, :][None, :]).sum(-1, keepdims=True))
    als_ref[...] = jnp.concatenate(als, axis=-1)
    ald_ref[...] = jnp.concatenate(ald, axis=-1)


def _stage_b(agg, b1g, W2, as2, ad2):
    return pl.pallas_call(
        _kb_body,
        grid=(N // BN,),
        in_specs=[
            pl.BlockSpec((BN, HEADS * HID), lambda i: (i, 0)),
            pl.BlockSpec((1, HEADS * HID), lambda i: (0, 0)),
            pl.BlockSpec((HEADS * HID, HEADS * OUT), lambda i: (0, 0)),
            pl.BlockSpec((HEADS, OUT), lambda i: (0, 0)),
            pl.BlockSpec((HEADS, OUT), lambda i: (0, 0)),
        ],
        out_specs=[
            pl.BlockSpec((BN, HEADS * HID), lambda i: (i, 0)),
            pl.BlockSpec((BN, HEADS * OUT), lambda i: (i, 0)),
            pl.BlockSpec((BN, HEADS), lambda i: (i, 0)),
            pl.BlockSpec((BN, HEADS), lambda i: (i, 0)),
        ],
        out_shape=[
            jax.ShapeDtypeStruct((N, HEADS * HID), jnp.float32),
            jax.ShapeDtypeStruct((N, HEADS * OUT), jnp.float32),
            jax.ShapeDtypeStruct((N, HEADS), jnp.float32),
            jax.ShapeDtypeStruct((N, HEADS), jnp.float32),
        ],
    )(agg, b1g.reshape(1, -1), W2, as2, ad2)


def _ln(x, g, b):
    m = x.mean(-1, keepdims=True)
    v = ((x - m) ** 2).mean(-1, keepdims=True)
    return (x - m) * jax.lax.rsqrt(v + 1e-5) * g + b


def _kc_body(ht_ref, pe_ref, wv_ref, bv_ref, wo_ref, bo_ref, g_ref, b_ref,
             wf1_ref, bf1_ref, wf2_ref, bf2_ref, wc_ref, bc_ref, out_ref):
    u = ht_ref[...] + pe_ref[...]
    v = jnp.dot(u, wv_ref[...], preferred_element_type=jnp.float32) + bv_ref[...]
    attn = jnp.dot(v, wo_ref[...], preferred_element_type=jnp.float32) + bo_ref[...]
    g = g_ref[...]
    b = b_ref[...]
    y = _ln(u + attn, g, b)
    f = jnp.dot(y, wf1_ref[...], preferred_element_type=jnp.float32) + bf1_ref[...]
    f = jnp.maximum(f, 0.0)
    f = jnp.dot(f, wf2_ref[...], preferred_element_type=jnp.float32) + bf2_ref[...]
    y = _ln(y + f, g, b)
    out_ref[...] = jnp.dot(y, wc_ref[...], preferred_element_type=jnp.float32) + bc_ref[...]


def _stage_c(h_t, Wv, bv, Wo, bo, ln_g, ln_b, Wf1, bf1, Wf2, bf2, Wc, bc):
    pe = jnp.asarray(_PE4).reshape(1, OUT)
    row = lambda a: a.reshape(1, -1)
    full = lambda shape: pl.BlockSpec(shape, lambda i: tuple(0 for _ in shape))
    return pl.pallas_call(
        _kc_body,
        grid=(N // BN,),
        in_specs=[
            pl.BlockSpec((BN, OUT), lambda i: (i, 0)),
            full((1, OUT)),
            full((OUT, OUT)), full((1, OUT)),
            full((OUT, OUT)), full((1, OUT)),
            full((1, OUT)), full((1, OUT)),
            full((OUT, 4 * OUT)), full((1, 4 * OUT)),
            full((4 * OUT, OUT)), full((1, OUT)),
            full((OUT, NCLS)), full((1, NCLS)),
        ],
        out_specs=pl.BlockSpec((BN, NCLS), lambda i: (i, 0)),
        out_shape=jax.ShapeDtypeStruct((N, NCLS), jnp.float32),
    )(h_t, pe, Wv, row(bv), Wo, row(bo), row(ln_g), row(ln_b),
      Wf1, row(bf1), Wf2, row(bf2), Wc, row(bc))


def _edge_aggregate(h0, als, ald, src, dst, keep):
    al = als[src] + ald[dst]
    al = jnp.where(al >= 0, al, 0.2 * al)
    ex = jnp.where(keep[:, None], jnp.exp(al), 0.0)
    den = jax.ops.segment_sum(ex, dst, num_segments=N)
    w = ex / (den[dst] + 1e-16)
    hr = h0.reshape(N, HEADS, -1)
    agg = jax.ops.segment_sum(hr[src] * w[:, :, None], dst, num_segments=N)
    return agg.reshape(N, -1)


def kernel(x, edge_index, time_step, W1, as1, ad1, b1g, W2, as2, ad2, b2g,
           Wqkv, bqkv, Wo, bo, ln_g, ln_b, Wf1, bf1, Wf2, bf2, Wc, bc):
    src0, dst0 = edge_index[0], edge_index[1]
    loops = jnp.arange(N, dtype=src0.dtype)
    src = jnp.concatenate([src0, dst0, loops])
    dst = jnp.concatenate([dst0, src0, loops])
    key = dst * N + src
    order = jnp.argsort(key)
    src = src[order]
    dst = dst[order]
    key_s = key[order]
    first = jnp.concatenate([jnp.ones((1,), bool), key_s[1:] != key_s[:-1]])
    keep = first & (time_step[src] == time_step[dst])

    h0, als1, ald1 = _stage_a(x, W1, as1, ad1, D_IN, HEADS * HID)
    agg1 = _edge_aggregate(h0, als1, ald1, src, dst, keep)
    h, h1, als2, ald2 = _stage_b(agg1, b1g, W2, as2, ad2)
    agg2 = _edge_aggregate(h1, als2, ald2, src, dst, keep)
    h_t = agg2.reshape(N, HEADS, OUT).mean(axis=1) + b2g

    Wv = Wqkv[:, 2 * OUT:]
    bv = bqkv[2 * OUT:]
    return _stage_c(h_t, Wv, bv, Wo, bo, ln_g, ln_b, Wf1, bf1, Wf2, bf2, Wc, bc)
